# baseline (speedup 1.0000x reference)
import jax
import jax.numpy as jnp
from jax import lax
from jax.experimental import pallas as pl
from jax.experimental.pallas import tpu as pltpu


def _som_body(x_ref, w_ref, wout_ref, winner_ref):
    x = x_ref[...]
    w = w_ref[...]
    xw = lax.dot_general(x, w, (((1,), (1,)), ((), ())),
                         preferred_element_type=jnp.float32)
    xn = jnp.sum(x * x, axis=1, keepdims=True)
    wn = jnp.sum(w * w, axis=1, keepdims=True)
    d2 = xn - 2.0 * xw + wn.T
    winner_ref[...] = jnp.argmin(d2, axis=1).astype(jnp.int32)
    wout_ref[...] = w


def kernel(inputs, W):
    B, D = inputs.shape
    M, _ = W.shape
    wout, _winner = pl.pallas_call(
        _som_body,
        out_shape=(
            jax.ShapeDtypeStruct((M, D), W.dtype),
            jax.ShapeDtypeStruct((B,), jnp.int32),
        ),
    )(inputs, W)
    return wout
